# Initial kernel scaffold; baseline (speedup 1.0000x reference)
import jax
import jax.numpy as jnp
from jax.experimental import pallas as pl


def kernel(x, edge_index, W1, b1, W2, b2, W3, b3):
    raise NotImplementedError("write your pallas kernel here")


# speedup vs baseline: 29.0947x; 29.0947x over previous
import functools

import jax
import jax.numpy as jnp
from jax import lax
from jax.experimental import pallas as pl
from jax.experimental.pallas import tpu as pltpu
from jax.experimental.pallas import tpu_sc as plsc

NC = 2
NS = 16
D16 = 16
K_INNER = 8
BLK = 2048


def _edge_pass(npad, nbt, do_gather):
  mesh = plsc.VectorSubcoreMesh(core_axis_name="c", subcore_axis_name="s")
  rows_per_tile = npad // NS
  n_dump = rows_per_tile // 128

  def body(rowidx, colidx, table, zeros, out, acc, ridx, cidx, gbuf, sem):
    cid = lax.axis_index("c")
    sid = lax.axis_index("s")
    row0 = sid * rows_per_tile
    pltpu.sync_copy(zeros, gbuf.at[0])

    @pl.loop(0, n_dump)
    def _zero(i):
      pltpu.sync_copy(gbuf.at[0], acc.at[pl.ds(row0 + i * 128, 128)])

    if not do_gather:
      for j in range(K_INNER):
        pltpu.sync_copy(table.at[pl.ds(0, 128)], gbuf.at[j])
    plsc.subcore_barrier()

    jb0 = sid * nbt

    @pl.loop(0, nbt // K_INNER)
    def _edges(i):
      base = jb0 + i * K_INNER
      pltpu.sync_copy(colidx.at[cid, pl.ds(base, K_INNER)], cidx)
      if do_gather:
        pltpu.sync_copy(rowidx.at[cid, pl.ds(base, K_INNER)], ridx)
        cps = [
            pltpu.async_copy(table.at[ridx.at[j]], gbuf.at[j], sem)
            for j in range(K_INNER)
        ]
        for c in cps:
          c.wait()
      for j in range(K_INNER):
        pltpu.sync_copy(gbuf.at[j], acc.at[cidx.at[j]], add=True)

    plsc.subcore_barrier()

    @pl.loop(0, n_dump)
    def _dump(i):
      r = row0 + i * 128
      pltpu.sync_copy(acc.at[pl.ds(r, 128)], gbuf.at[0])
      pltpu.sync_copy(gbuf.at[0], out.at[cid, pl.ds(r, 128)])

  return pl.kernel(
      body,
      out_type=jax.ShapeDtypeStruct((NC, npad, D16), jnp.float32),
      mesh=mesh,
      compiler_params=pltpu.CompilerParams(use_tc_tiling_on_sc=False),
      scratch_types=[
          pltpu.VMEM_SHARED((npad, D16), jnp.float32),
          pltpu.VMEM((K_INNER, 128), jnp.int32),
          pltpu.VMEM((K_INNER, 128), jnp.int32),
          pltpu.VMEM((K_INNER, 128, D16), jnp.float32),
          pltpu.SemaphoreType.DMA,
      ],
  )


def _prep_body(cnt_ref, x_ref, z1_ref, dinv_ref):
  cnt = cnt_ref[...]
  deg = cnt[0, :, :1] + cnt[1, :, :1] + 1.0
  dinv = lax.rsqrt(deg)
  dinv_ref[...] = jnp.broadcast_to(dinv, x_ref.shape)
  z1_ref[...] = x_ref[...] * dinv


def _mid_body(s1_ref, z1_ref, dinv_ref, w1_ref, b1_ref, w2_ref, z2_ref):
  s1 = s1_ref[...]
  dinv = dinv_ref[...]
  prop1 = dinv * (s1[0] + s1[1] + z1_ref[...])
  h1 = jnp.maximum(
      jnp.dot(prop1, w1_ref[...], preferred_element_type=jnp.float32)
      + b1_ref[...], 0.0)
  y2 = jnp.dot(h1, w2_ref[...], preferred_element_type=jnp.float32)
  z2 = y2 * dinv[:, :1]
  z2_ref[0] = z2[:, :D16]
  z2_ref[1] = z2[:, D16:]


def _fin_body(n_nodes, s2_ref, z2_ref, dinv_ref, b2_ref, w3_ref, b3_ref,
              out_ref, acc_ref):
  i = pl.program_id(0)

  @pl.when(i == 0)
  def _():
    acc_ref[...] = jnp.zeros_like(acc_ref)

  s2 = s2_ref[...]
  z2 = z2_ref[...]
  dinv = dinv_ref[...]
  b2 = b2_ref[...]
  h2a = jnp.maximum(dinv * (s2[0] + z2[0]) + b2[:, :D16], 0.0)
  h2b = jnp.maximum(dinv * (s2[1] + z2[1]) + b2[:, D16:], 0.0)
  rows = i * BLK + lax.broadcasted_iota(jnp.int32, (BLK, 1), 0)
  m = jnp.where(rows < n_nodes, 1.0, 0.0)
  acc_ref[...] += jnp.concatenate(
      [jnp.sum(h2a * m, axis=0, keepdims=True),
       jnp.sum(h2b * m, axis=0, keepdims=True)], axis=1)

  @pl.when(i == pl.num_programs(0) - 1)
  def _():
    g = acc_ref[...] * (1.0 / n_nodes)
    logit = jnp.dot(g, w3_ref[...], preferred_element_type=jnp.float32)
    logit = logit + b3_ref[...]
    out_ref[...] = 1.0 / (1.0 + jnp.exp(-logit))


def kernel(x, edge_index, W1, b1, W2, b2, W3, b3):
  n = x.shape[0]
  e = edge_index.shape[1]
  npad = -(-n // (NS * 128)) * (NS * 128)
  grid = npad // BLK
  equant = NC * NS * 128 * K_INNER
  epad = -(-e // equant) * equant
  nbt_half = epad // NC // 128 // NS
  nbt_full = epad // 128 // NS

  i32 = jnp.int32
  f32 = jnp.float32
  rowp = jnp.concatenate([edge_index[0], jnp.full((epad - e,), n, i32)])
  colp = jnp.concatenate([edge_index[1], jnp.full((epad - e,), n, i32)])
  idx1r = rowp.reshape(NC, -1, 128)
  idx1c = colp.reshape(NC, -1, 128)
  idx2r = jnp.stack([rowp, rowp + npad]).reshape(NC, -1, 128)
  idx2c = jnp.stack([colp, colp]).reshape(NC, -1, 128)

  xpad = jnp.zeros((npad, D16), f32).at[:n, :4].set(x)
  zeros128 = jnp.zeros((128, D16), f32)
  ones_tab = jnp.zeros((128, D16), f32).at[:, 0].set(1.0)
  w1p = jnp.zeros((D16, 64), f32).at[:4].set(W1)

  cnt = _edge_pass(npad, nbt_half, False)(idx1c, idx1c, ones_tab, zeros128)

  z1, dinv16 = pl.pallas_call(
      _prep_body,
      grid=(grid,),
      in_specs=[
          pl.BlockSpec((NC, BLK, D16), lambda i: (0, i, 0)),
          pl.BlockSpec((BLK, D16), lambda i: (i, 0)),
      ],
      out_specs=[
          pl.BlockSpec((BLK, D16), lambda i: (i, 0)),
          pl.BlockSpec((BLK, D16), lambda i: (i, 0)),
      ],
      out_shape=[
          jax.ShapeDtypeStruct((npad, D16), f32),
          jax.ShapeDtypeStruct((npad, D16), f32),
      ],
  )(cnt, xpad)

  s1 = _edge_pass(npad, nbt_half, True)(idx1r, idx1c, z1, zeros128)

  z2 = pl.pallas_call(
      _mid_body,
      grid=(grid,),
      in_specs=[
          pl.BlockSpec((NC, BLK, D16), lambda i: (0, i, 0)),
          pl.BlockSpec((BLK, D16), lambda i: (i, 0)),
          pl.BlockSpec((BLK, D16), lambda i: (i, 0)),
          pl.BlockSpec((D16, 64), lambda i: (0, 0)),
          pl.BlockSpec((1, 64), lambda i: (0, 0)),
          pl.BlockSpec((64, 32), lambda i: (0, 0)),
      ],
      out_specs=pl.BlockSpec((NC, BLK, D16), lambda i: (0, i, 0)),
      out_shape=jax.ShapeDtypeStruct((NC, npad, D16), f32),
  )(s1, z1, dinv16, w1p, b1.reshape(1, 64), W2)

  tab2 = z2.reshape(NC * npad, D16)
  s2 = _edge_pass(npad, nbt_full, True)(idx2r, idx2c, tab2, zeros128)

  out = pl.pallas_call(
      functools.partial(_fin_body, n),
      grid=(grid,),
      in_specs=[
          pl.BlockSpec((NC, BLK, D16), lambda i: (0, i, 0)),
          pl.BlockSpec((NC, BLK, D16), lambda i: (0, i, 0)),
          pl.BlockSpec((BLK, D16), lambda i: (i, 0)),
          pl.BlockSpec((1, 32), lambda i: (0, 0)),
          pl.BlockSpec((32, 1), lambda i: (0, 0)),
          pl.BlockSpec((1, 1), lambda i: (0, 0)),
      ],
      out_specs=pl.BlockSpec((1, 1), lambda i: (0, 0)),
      out_shape=jax.ShapeDtypeStruct((1, 1), f32),
      scratch_shapes=[pltpu.VMEM((1, 32), f32)],
  )(s2, z2, dinv16, b2.reshape(1, 32), W3, b3.reshape(1, 1))
  return out


# speedup vs baseline: 32.3733x; 1.1127x over previous
import functools

import jax
import jax.numpy as jnp
from jax import lax
from jax.experimental import pallas as pl
from jax.experimental.pallas import tpu as pltpu
from jax.experimental.pallas import tpu_sc as plsc

NC = 2
NS = 16
D16 = 16
K_INNER = 8
BLK = 2048

_SC_PARAMS = pltpu.CompilerParams(use_tc_tiling_on_sc=False)
_MESH = plsc.VectorSubcoreMesh(core_axis_name="c", subcore_axis_name="s")


def _edge_pass(npad, nbt):
  rows_per_tile = npad // NS
  n_dump = rows_per_tile // 128

  def body(idx, table, zeros, out, acc, ibuf, gbuf, sem):
    cid = lax.axis_index("c")
    sid = lax.axis_index("s")
    row0 = sid * rows_per_tile
    pltpu.sync_copy(zeros, gbuf.at[0])

    @pl.loop(0, n_dump)
    def _zero(i):
      pltpu.sync_copy(gbuf.at[0], acc.at[pl.ds(row0 + i * 128, 128)])

    plsc.subcore_barrier()
    jb0 = sid * nbt

    @pl.loop(0, nbt // K_INNER)
    def _edges(i):
      pltpu.sync_copy(idx.at[cid, pl.ds(jb0 + i * K_INNER, K_INNER)], ibuf)
      cps = [
          pltpu.async_copy(table.at[ibuf.at[j, 0]], gbuf.at[j], sem)
          for j in range(K_INNER)
      ]
      for j in range(K_INNER):
        cps[j].wait()
        pltpu.sync_copy(gbuf.at[j], acc.at[ibuf.at[j, 1]], add=True)

    plsc.subcore_barrier()

    @pl.loop(0, n_dump)
    def _dump(i):
      r = row0 + i * 128
      pltpu.sync_copy(acc.at[pl.ds(r, 128)], gbuf.at[0])
      pltpu.sync_copy(gbuf.at[0], out.at[cid, pl.ds(r, 128)])

  return pl.kernel(
      body,
      out_type=jax.ShapeDtypeStruct((NC, npad, D16), jnp.float32),
      mesh=_MESH,
      compiler_params=_SC_PARAMS,
      scratch_types=[
          pltpu.VMEM_SHARED((npad, D16), jnp.float32),
          pltpu.VMEM((K_INNER, 2, 128), jnp.int32),
          pltpu.VMEM((K_INNER, 128, D16), jnp.float32),
          pltpu.SemaphoreType.DMA,
      ],
  )


def _deg_pass(npad, nbt):
  rows_per_tile = npad // NS

  def body(cidx, ones, zrows, out, acc, ibuf, onebuf, rbuf):
    cid = lax.axis_index("c")
    sid = lax.axis_index("s")
    row0 = sid * rows_per_tile
    pltpu.sync_copy(zrows, rbuf)
    pltpu.sync_copy(rbuf, acc.at[pl.ds(row0, rows_per_tile)])
    pltpu.sync_copy(ones, onebuf)
    plsc.subcore_barrier()
    jb0 = sid * nbt

    @pl.loop(0, nbt // K_INNER)
    def _edges(i):
      pltpu.sync_copy(cidx.at[cid, pl.ds(jb0 + i * K_INNER, K_INNER)], ibuf)
      for j in range(K_INNER):
        pltpu.sync_copy(onebuf, acc.at[ibuf.at[j]], add=True)

    plsc.subcore_barrier()
    pltpu.sync_copy(acc.at[pl.ds(row0, rows_per_tile)], rbuf)
    pltpu.sync_copy(rbuf, out.at[cid, pl.ds(row0, rows_per_tile)])

  return pl.kernel(
      body,
      out_type=jax.ShapeDtypeStruct((NC, npad), jnp.float32),
      mesh=_MESH,
      compiler_params=_SC_PARAMS,
      scratch_types=[
          pltpu.VMEM_SHARED((npad,), jnp.float32),
          pltpu.VMEM((K_INNER, 128), jnp.int32),
          pltpu.VMEM((128,), jnp.float32),
          pltpu.VMEM((rows_per_tile,), jnp.float32),
      ],
  )


def _prep_body(cnt_ref, x_ref, z1_ref, dinv_ref):
  cnt = cnt_ref[...]
  deg = cnt[0] + cnt[1] + 1.0
  dinv = lax.rsqrt(deg)
  dinv_ref[...] = jnp.broadcast_to(dinv, x_ref.shape)
  z1_ref[...] = x_ref[...] * dinv


def _mid_body(s1_ref, z1_ref, dinv_ref, w1_ref, b1_ref, w2_ref, z2_ref):
  s1 = s1_ref[...]
  dinv = dinv_ref[...]
  prop1 = dinv * (s1[0] + s1[1] + z1_ref[...])
  h1 = jnp.maximum(
      jnp.dot(prop1, w1_ref[...], preferred_element_type=jnp.float32)
      + b1_ref[...], 0.0)
  y2 = jnp.dot(h1, w2_ref[...], preferred_element_type=jnp.float32)
  z2 = y2 * dinv[:, :1]
  z2_ref[0] = z2[:, :D16]
  z2_ref[1] = z2[:, D16:]


def _fin_body(n_nodes, s2_ref, z2_ref, dinv_ref, b2_ref, w3_ref, b3_ref,
              out_ref, acc_ref):
  i = pl.program_id(0)

  @pl.when(i == 0)
  def _():
    acc_ref[...] = jnp.zeros_like(acc_ref)

  s2 = s2_ref[...]
  z2 = z2_ref[...]
  dinv = dinv_ref[...]
  b2 = b2_ref[...]
  h2a = jnp.maximum(dinv * (s2[0] + z2[0]) + b2[:, :D16], 0.0)
  h2b = jnp.maximum(dinv * (s2[1] + z2[1]) + b2[:, D16:], 0.0)
  rows = i * BLK + lax.broadcasted_iota(jnp.int32, (BLK, 1), 0)
  m = jnp.where(rows < n_nodes, 1.0, 0.0)
  acc_ref[...] += jnp.concatenate(
      [jnp.sum(h2a * m, axis=0, keepdims=True),
       jnp.sum(h2b * m, axis=0, keepdims=True)], axis=1)

  @pl.when(i == pl.num_programs(0) - 1)
  def _():
    g = acc_ref[...] * (1.0 / n_nodes)
    logit = jnp.dot(g, w3_ref[...], preferred_element_type=jnp.float32)
    logit = logit + b3_ref[...]
    out_ref[...] = 1.0 / (1.0 + jnp.exp(-logit))


def kernel(x, edge_index, W1, b1, W2, b2, W3, b3):
  n = x.shape[0]
  e = edge_index.shape[1]
  npad = -(-n // (NS * 128)) * (NS * 128)
  grid = npad // BLK
  equant = NC * NS * 128 * K_INNER
  epad = -(-e // equant) * equant
  nbt_half = epad // NC // 128 // NS
  nbt_full = epad // 128 // NS

  i32 = jnp.int32
  f32 = jnp.float32
  rowp = jnp.concatenate([edge_index[0], jnp.full((epad - e,), n, i32)])
  colp = jnp.concatenate([edge_index[1], jnp.full((epad - e,), n, i32)])
  r128 = rowp.reshape(-1, 128)
  c128 = colp.reshape(-1, 128)
  idx1 = jnp.stack([r128, c128], axis=1).reshape(NC, -1, 2, 128)
  idx2 = jnp.concatenate([
      jnp.stack([r128, c128], axis=1),
      jnp.stack([r128 + npad, c128], axis=1),
  ]).reshape(NC, -1, 2, 128)
  cidx1 = colp.reshape(NC, -1, 128)

  xpad = jnp.zeros((npad, D16), f32).at[:n, :4].set(x)
  zeros128 = jnp.zeros((128, D16), f32)
  ones128 = jnp.ones((128,), f32)
  zrows = jnp.zeros((npad // NS,), f32)
  w1p = jnp.zeros((D16, 64), f32).at[:4].set(W1)

  cnt = _deg_pass(npad, nbt_half)(cidx1, ones128, zrows)

  z1, dinv16 = pl.pallas_call(
      _prep_body,
      grid=(grid,),
      in_specs=[
          pl.BlockSpec((NC, BLK, 1), lambda i: (0, i, 0)),
          pl.BlockSpec((BLK, D16), lambda i: (i, 0)),
      ],
      out_specs=[
          pl.BlockSpec((BLK, D16), lambda i: (i, 0)),
          pl.BlockSpec((BLK, D16), lambda i: (i, 0)),
      ],
      out_shape=[
          jax.ShapeDtypeStruct((npad, D16), f32),
          jax.ShapeDtypeStruct((npad, D16), f32),
      ],
  )(cnt.reshape(NC, npad, 1), xpad)

  s1 = _edge_pass(npad, nbt_half)(idx1, z1, zeros128)

  z2 = pl.pallas_call(
      _mid_body,
      grid=(grid,),
      in_specs=[
          pl.BlockSpec((NC, BLK, D16), lambda i: (0, i, 0)),
          pl.BlockSpec((BLK, D16), lambda i: (i, 0)),
          pl.BlockSpec((BLK, D16), lambda i: (i, 0)),
          pl.BlockSpec((D16, 64), lambda i: (0, 0)),
          pl.BlockSpec((1, 64), lambda i: (0, 0)),
          pl.BlockSpec((64, 32), lambda i: (0, 0)),
      ],
      out_specs=pl.BlockSpec((NC, BLK, D16), lambda i: (0, i, 0)),
      out_shape=jax.ShapeDtypeStruct((NC, npad, D16), f32),
  )(s1, z1, dinv16, w1p, b1.reshape(1, 64), W2)

  tab2 = z2.reshape(NC * npad, D16)
  s2 = _edge_pass(npad, nbt_full)(idx2, tab2, zeros128)

  out = pl.pallas_call(
      functools.partial(_fin_body, n),
      grid=(grid,),
      in_specs=[
          pl.BlockSpec((NC, BLK, D16), lambda i: (0, i, 0)),
          pl.BlockSpec((NC, BLK, D16), lambda i: (0, i, 0)),
          pl.BlockSpec((BLK, D16), lambda i: (i, 0)),
          pl.BlockSpec((1, 32), lambda i: (0, 0)),
          pl.BlockSpec((32, 1), lambda i: (0, 0)),
          pl.BlockSpec((1, 1), lambda i: (0, 0)),
      ],
      out_specs=pl.BlockSpec((1, 1), lambda i: (0, 0)),
      out_shape=jax.ShapeDtypeStruct((1, 1), f32),
      scratch_shapes=[pltpu.VMEM((1, 32), f32)],
  )(s2, z2, dinv16, b2.reshape(1, 32), W3, b3.reshape(1, 1))
  return out


# speedup vs baseline: 36.3484x; 1.1228x over previous
import functools

import jax
import jax.numpy as jnp
from jax import lax
from jax.experimental import pallas as pl
from jax.experimental.pallas import tpu as pltpu
from jax.experimental.pallas import tpu_sc as plsc

NC = 2
NS = 16
D16 = 16
K_INNER = 8
K_PIPE = 4
BLK = 2048

_SC_PARAMS = pltpu.CompilerParams(use_tc_tiling_on_sc=False)
_MESH = plsc.VectorSubcoreMesh(core_axis_name="c", subcore_axis_name="s")


def _edge_pass(npad, nbt):
  rows_per_tile = npad // NS
  n_dump = rows_per_tile // 128

  def body(idx, table, zeros, out, acc, ibuf, gbuf, sem, isem):
    cid = lax.axis_index("c")
    sid = lax.axis_index("s")
    row0 = sid * rows_per_tile
    pltpu.sync_copy(zeros, gbuf.at[0, 0])

    @pl.loop(0, n_dump)
    def _zero(i):
      pltpu.sync_copy(gbuf.at[0, 0], acc.at[pl.ds(row0 + i * 128, 128)])

    plsc.subcore_barrier()
    jb0 = sid * nbt
    n_it = nbt // K_PIPE

    pltpu.sync_copy(idx.at[cid, pl.ds(jb0, K_PIPE)], ibuf.at[0])
    for j in range(K_PIPE):
      pltpu.async_copy(table.at[ibuf.at[0, j, 0]], gbuf.at[0, j], sem)
    if n_it > 1:
      pltpu.async_copy(
          idx.at[cid, pl.ds(jb0 + K_PIPE, K_PIPE)], ibuf.at[1], isem)

    @pl.loop(0, n_it)
    def _edges(i):
      p = lax.rem(i, 2)
      q = lax.rem(i, 3)
      q1 = lax.rem(i + 1, 3)
      q2 = lax.rem(i + 2, 3)

      @pl.when(i + 1 < n_it)
      def _fire_next():
        pltpu.make_async_copy(
            idx.at[cid, pl.ds(jb0 + (i + 1) * K_PIPE, K_PIPE)],
            ibuf.at[q1], isem).wait()
        for j in range(K_PIPE):
          pltpu.async_copy(
              table.at[ibuf.at[q1, j, 0]], gbuf.at[1 - p, j], sem)

      @pl.when(i + 2 < n_it)
      def _prefetch_idx():
        pltpu.async_copy(
            idx.at[cid, pl.ds(jb0 + (i + 2) * K_PIPE, K_PIPE)],
            ibuf.at[q2], isem)

      for j in range(K_PIPE):
        pltpu.make_async_copy(
            table.at[ibuf.at[q, j, 0]], gbuf.at[p, j], sem).wait()
        pltpu.sync_copy(gbuf.at[p, j], acc.at[ibuf.at[q, j, 1]], add=True)

    plsc.subcore_barrier()

    @pl.loop(0, n_dump)
    def _dump(i):
      r = row0 + i * 128
      pltpu.sync_copy(acc.at[pl.ds(r, 128)], gbuf.at[0, 0])
      pltpu.sync_copy(gbuf.at[0, 0], out.at[cid, pl.ds(r, 128)])

  return pl.kernel(
      body,
      out_type=jax.ShapeDtypeStruct((NC, npad, D16), jnp.float32),
      mesh=_MESH,
      compiler_params=_SC_PARAMS,
      scratch_types=[
          pltpu.VMEM_SHARED((npad, D16), jnp.float32),
          pltpu.VMEM((3, K_PIPE, 2, 128), jnp.int32),
          pltpu.VMEM((2, K_PIPE, 128, D16), jnp.float32),
          pltpu.SemaphoreType.DMA,
          pltpu.SemaphoreType.DMA,
      ],
  )


def _deg_pass(npad, nbt):
  rows_per_tile = npad // NS

  def body(cidx, ones, zrows, out, acc, ibuf, onebuf, rbuf):
    cid = lax.axis_index("c")
    sid = lax.axis_index("s")
    row0 = sid * rows_per_tile
    pltpu.sync_copy(zrows, rbuf)
    pltpu.sync_copy(rbuf, acc.at[pl.ds(row0, rows_per_tile)])
    pltpu.sync_copy(ones, onebuf)
    plsc.subcore_barrier()
    jb0 = sid * nbt

    @pl.loop(0, nbt // K_INNER)
    def _edges(i):
      pltpu.sync_copy(cidx.at[cid, pl.ds(jb0 + i * K_INNER, K_INNER)], ibuf)
      for j in range(K_INNER):
        pltpu.sync_copy(onebuf, acc.at[ibuf.at[j]], add=True)

    plsc.subcore_barrier()
    pltpu.sync_copy(acc.at[pl.ds(row0, rows_per_tile)], rbuf)
    pltpu.sync_copy(rbuf, out.at[cid, pl.ds(row0, rows_per_tile)])

  return pl.kernel(
      body,
      out_type=jax.ShapeDtypeStruct((NC, npad), jnp.float32),
      mesh=_MESH,
      compiler_params=_SC_PARAMS,
      scratch_types=[
          pltpu.VMEM_SHARED((npad,), jnp.float32),
          pltpu.VMEM((K_INNER, 128), jnp.int32),
          pltpu.VMEM((128,), jnp.float32),
          pltpu.VMEM((rows_per_tile,), jnp.float32),
      ],
  )


def _prep_body(cnt_ref, x_ref, z1_ref, dinv_ref):
  cnt = cnt_ref[...]
  deg = cnt[0] + cnt[1] + 1.0
  dinv = lax.rsqrt(deg)
  dinv_ref[...] = jnp.broadcast_to(dinv, x_ref.shape)
  z1_ref[...] = x_ref[...] * dinv


def _mid_body(s1_ref, z1_ref, dinv_ref, w1_ref, b1_ref, w2_ref, z2_ref):
  s1 = s1_ref[...]
  dinv = dinv_ref[...]
  prop1 = dinv * (s1[0] + s1[1] + z1_ref[...])
  h1 = jnp.maximum(
      jnp.dot(prop1, w1_ref[...], preferred_element_type=jnp.float32)
      + b1_ref[...], 0.0)
  y2 = jnp.dot(h1, w2_ref[...], preferred_element_type=jnp.float32)
  z2 = y2 * dinv[:, :1]
  z2_ref[0] = z2[:, :D16]
  z2_ref[1] = z2[:, D16:]


def _fin_body(n_nodes, s2_ref, z2_ref, dinv_ref, b2_ref, w3_ref, b3_ref,
              out_ref, acc_ref):
  i = pl.program_id(0)

  @pl.when(i == 0)
  def _():
    acc_ref[...] = jnp.zeros_like(acc_ref)

  s2 = s2_ref[...]
  z2 = z2_ref[...]
  dinv = dinv_ref[...]
  b2 = b2_ref[...]
  h2a = jnp.maximum(dinv * (s2[0] + z2[0]) + b2[:, :D16], 0.0)
  h2b = jnp.maximum(dinv * (s2[1] + z2[1]) + b2[:, D16:], 0.0)
  rows = i * BLK + lax.broadcasted_iota(jnp.int32, (BLK, 1), 0)
  m = jnp.where(rows < n_nodes, 1.0, 0.0)
  acc_ref[...] += jnp.concatenate(
      [jnp.sum(h2a * m, axis=0, keepdims=True),
       jnp.sum(h2b * m, axis=0, keepdims=True)], axis=1)

  @pl.when(i == pl.num_programs(0) - 1)
  def _():
    g = acc_ref[...] * (1.0 / n_nodes)
    logit = jnp.dot(g, w3_ref[...], preferred_element_type=jnp.float32)
    logit = logit + b3_ref[...]
    out_ref[...] = 1.0 / (1.0 + jnp.exp(-logit))


def kernel(x, edge_index, W1, b1, W2, b2, W3, b3):
  n = x.shape[0]
  e = edge_index.shape[1]
  npad = -(-n // (NS * 128)) * (NS * 128)
  grid = npad // BLK
  equant = NC * NS * 128 * K_INNER
  epad = -(-e // equant) * equant
  nbt_half = epad // NC // 128 // NS
  nbt_full = epad // 128 // NS

  i32 = jnp.int32
  f32 = jnp.float32
  rowp = jnp.concatenate([edge_index[0], jnp.full((epad - e,), n, i32)])
  colp = jnp.concatenate([edge_index[1], jnp.full((epad - e,), n, i32)])
  r128 = rowp.reshape(-1, 128)
  c128 = colp.reshape(-1, 128)
  base = jnp.stack([r128, c128], axis=1)
  idx1 = base.reshape(NC, -1, 2, 128)
  idx2 = jnp.concatenate(
      [base, jnp.stack([r128 + npad, c128], axis=1)]).reshape(NC, -1, 2, 128)
  cidx1 = colp.reshape(NC, -1, 128)

  xpad = jnp.zeros((npad, D16), f32).at[:n, :4].set(x)
  zeros128 = jnp.zeros((128, D16), f32)
  ones128 = jnp.ones((128,), f32)
  zrows = jnp.zeros((npad // NS,), f32)
  w1p = jnp.zeros((D16, 64), f32).at[:4].set(W1)

  cnt = _deg_pass(npad, nbt_half)(cidx1, ones128, zrows)

  z1, dinv16 = pl.pallas_call(
      _prep_body,
      grid=(grid,),
      in_specs=[
          pl.BlockSpec((NC, BLK, 1), lambda i: (0, i, 0)),
          pl.BlockSpec((BLK, D16), lambda i: (i, 0)),
      ],
      out_specs=[
          pl.BlockSpec((BLK, D16), lambda i: (i, 0)),
          pl.BlockSpec((BLK, D16), lambda i: (i, 0)),
      ],
      out_shape=[
          jax.ShapeDtypeStruct((npad, D16), f32),
          jax.ShapeDtypeStruct((npad, D16), f32),
      ],
  )(cnt.reshape(NC, npad, 1), xpad)

  s1 = _edge_pass(npad, nbt_half)(idx1, z1, zeros128)

  z2 = pl.pallas_call(
      _mid_body,
      grid=(grid,),
      in_specs=[
          pl.BlockSpec((NC, BLK, D16), lambda i: (0, i, 0)),
          pl.BlockSpec((BLK, D16), lambda i: (i, 0)),
          pl.BlockSpec((BLK, D16), lambda i: (i, 0)),
          pl.BlockSpec((D16, 64), lambda i: (0, 0)),
          pl.BlockSpec((1, 64), lambda i: (0, 0)),
          pl.BlockSpec((64, 32), lambda i: (0, 0)),
      ],
      out_specs=pl.BlockSpec((NC, BLK, D16), lambda i: (0, i, 0)),
      out_shape=jax.ShapeDtypeStruct((NC, npad, D16), f32),
  )(s1, z1, dinv16, w1p, b1.reshape(1, 64), W2)

  tab2 = z2.reshape(NC * npad, D16)
  s2 = _edge_pass(npad, nbt_full)(idx2, tab2, zeros128)

  out = pl.pallas_call(
      functools.partial(_fin_body, n),
      grid=(grid,),
      in_specs=[
          pl.BlockSpec((NC, BLK, D16), lambda i: (0, i, 0)),
          pl.BlockSpec((NC, BLK, D16), lambda i: (0, i, 0)),
          pl.BlockSpec((BLK, D16), lambda i: (i, 0)),
          pl.BlockSpec((1, 32), lambda i: (0, 0)),
          pl.BlockSpec((32, 1), lambda i: (0, 0)),
          pl.BlockSpec((1, 1), lambda i: (0, 0)),
      ],
      out_specs=pl.BlockSpec((1, 1), lambda i: (0, 0)),
      out_shape=jax.ShapeDtypeStruct((1, 1), f32),
      scratch_shapes=[pltpu.VMEM((1, 32), f32)],
  )(s2, z2, dinv16, b2.reshape(1, 32), W3, b3.reshape(1, 1))
  return out
